# Initial kernel scaffold; baseline (speedup 1.0000x reference)
import jax
import jax.numpy as jnp
from jax.experimental import pallas as pl


def kernel(x, W_add, b_add, w_relation, w_standard, bias, src, rel, dst):
    raise NotImplementedError("write your pallas kernel here")


# speedup vs baseline: 1.3874x; 1.3874x over previous
import functools

import jax
import jax.numpy as jnp
from jax import lax
from jax.experimental import pallas as pl
from jax.experimental.pallas import tpu as pltpu
from jax.experimental.pallas import tpu_sc as plsc

_NC = 2
_NS = 16
_NW = _NC * _NS
_L = 16

_CH = 128

_GATHER_DNUMS = lax.GatherDimensionNumbers(
    offset_dims=(), collapsed_slice_dims=(0,), start_index_map=(0,)
)


def _rotate(v, sh):
    idx = lax.rem(lax.iota(jnp.int32, _L) + sh, _L)
    return lax.gather(
        v, idx[:, None], _GATHER_DNUMS, slice_sizes=(1,),
        mode=lax.GatherScatterMode.PROMISE_IN_BOUNDS,
    )


def _lane_sum(v):
    for sh in (8, 4, 2, 1):
        v = v + _rotate(v, sh)
    return v


def _tc_embed(x, WaddT, badd_row):
    N, D = x.shape
    BLK = 1000
    assert N % BLK == 0

    def body(x_ref, w_ref, b_ref, o_ref):
        o_ref[...] = (
            jnp.dot(x_ref[...], w_ref[...], preferred_element_type=jnp.float32,
                    precision=lax.Precision.HIGHEST)
            + b_ref[...]
        )

    return pl.pallas_call(
        body,
        grid=(N // BLK,),
        in_specs=[
            pl.BlockSpec((BLK, D), lambda i: (i, 0)),
            pl.BlockSpec((D, D), lambda i: (0, 0)),
            pl.BlockSpec((1, D), lambda i: (0, 0)),
        ],
        out_specs=pl.BlockSpec((BLK, D), lambda i: (i, 0)),
        out_shape=jax.ShapeDtypeStruct((N, D), jnp.float32),
    )(x, WaddT, badd_row)


def _sc_score(emb, w_relation, w_standard, bias_vec, src_p, rel_p, dst_p,
              n_chunks):
    N, D = emb.shape
    R = w_relation.shape[0]
    E_pad = src_p.shape[0]
    mesh = plsc.VectorSubcoreMesh(core_axis_name="c", subcore_axis_name="s")

    @functools.partial(
        pl.kernel,
        out_type=jax.ShapeDtypeStruct((E_pad,), jnp.float32),
        mesh=mesh,
        scratch_types=[
            pltpu.VMEM((_CH,), jnp.int32),
            pltpu.VMEM((_CH,), jnp.int32),
            pltpu.VMEM((_CH,), jnp.int32),
            pltpu.VMEM((_CH, D), jnp.float32),
            pltpu.VMEM((_CH, D), jnp.float32),
            pltpu.VMEM((R, D), jnp.float32),
            pltpu.VMEM((R, 2 * D), jnp.float32),
            pltpu.VMEM((R,), jnp.float32),
            pltpu.VMEM((_CH,), jnp.float32),
            pltpu.SemaphoreType.DMA,
            pltpu.SemaphoreType.DMA,
        ],
    )
    def k(emb_hbm, wrel_hbm, wstd_hbm, bias_hbm, src_hbm, rel_hbm, dst_hbm,
          out_hbm, src_v, dst_v, rel_v, s_rows, o_rows, wrel_v, wstd_v,
          bias_v, score_v, sem1, sem2):
        wid = lax.axis_index("s") * _NC + lax.axis_index("c")
        pltpu.sync_copy(wrel_hbm, wrel_v)
        pltpu.sync_copy(wstd_hbm, wstd_v)
        pltpu.sync_copy(bias_hbm, bias_v)
        lanes = lax.iota(jnp.int32, _L)
        bias_reg = bias_v[pl.ds(0, _L)]

        def chunk_body(i, carry):
            base = (wid * n_chunks + i) * _CH
            pltpu.sync_copy(src_hbm.at[pl.ds(base, _CH)], src_v)
            pltpu.sync_copy(dst_hbm.at[pl.ds(base, _CH)], dst_v)
            pltpu.sync_copy(rel_hbm.at[pl.ds(base, _CH)], rel_v)
            cp1 = pltpu.async_copy(emb_hbm.at[src_v], s_rows, sem1)
            cp2 = pltpu.async_copy(emb_hbm.at[dst_v], o_rows, sem2)
            cp1.wait()
            cp2.wait()

            def group_body(g, carry2):
                score = jnp.zeros((_L,), jnp.float32)
                relg = rel_v[pl.ds(g * _L, _L)]
                for t in range(_L):
                    row = g * _L + t
                    relt = relg[t]
                    acc = jnp.where(lanes == relt, bias_reg, 0.0)
                    for c in range(D // _L):
                        s = s_rows[row, pl.ds(c * _L, _L)]
                        o = o_rows[row, pl.ds(c * _L, _L)]
                        r = wrel_v[relt, pl.ds(c * _L, _L)]
                        v1 = wstd_v[relt, pl.ds(c * _L, _L)]
                        v2 = wstd_v[relt, pl.ds(D + c * _L, _L)]
                        acc = acc + s * (r * o + v1) + o * v2
                    tot = _lane_sum(acc)
                    score = jnp.where(lanes == t, tot, score)
                score_v[pl.ds(g * _L, _L)] = score
                return carry2

            lax.fori_loop(0, _CH // _L, group_body, 0)
            pltpu.sync_copy(score_v, out_hbm.at[pl.ds(base, _CH)])
            return carry

        lax.fori_loop(0, n_chunks, chunk_body, 0)

    return k(emb, w_relation, w_standard, bias_vec, src_p, rel_p, dst_p)


def kernel(x, W_add, b_add, w_relation, w_standard, bias, src, rel, dst):
    N, D = x.shape
    R = w_relation.shape[0]
    E = src.shape[0]

    emb = _tc_embed(x, W_add.T, b_add.reshape(1, D))

    per_worker = -(-E // _NW)
    n_chunks = -(-per_worker // _CH)
    E_pad = _NW * n_chunks * _CH
    pad = E_pad - E
    src_p = jnp.pad(src, (0, pad))
    rel_p = jnp.pad(rel, (0, pad))
    dst_p = jnp.pad(dst, (0, pad))

    scores = _sc_score(emb, w_relation, w_standard, bias.reshape(R),
                       src_p, rel_p, dst_p, n_chunks)
    return scores[:E]


# speedup vs baseline: 2.1862x; 1.5758x over previous
import functools

import jax
import jax.numpy as jnp
from jax import lax
from jax.experimental import pallas as pl
from jax.experimental.pallas import tpu as pltpu
from jax.experimental.pallas import tpu_sc as plsc

_NC = 2
_NS = 16
_NW = _NC * _NS
_L = 16

_CH = 64
_RW = 256

_GATHER_DNUMS = lax.GatherDimensionNumbers(
    offset_dims=(), collapsed_slice_dims=(0,), start_index_map=(0,)
)


def _rotate(v, sh):
    idx = lax.rem(lax.iota(jnp.int32, _L) + sh, _L)
    return lax.gather(
        v, idx[:, None], _GATHER_DNUMS, slice_sizes=(1,),
        mode=lax.GatherScatterMode.PROMISE_IN_BOUNDS,
    )


def _lane_sum(v):
    for sh in (8, 4, 2, 1):
        v = v + _rotate(v, sh)
    return v


def _tc_embed(x, WaddT, badd_row, v1T, v2T, bias_row):
    N, D = x.shape
    R = v1T.shape[1]
    BLK = 1000
    assert N % BLK == 0

    def body(x_ref, w_ref, b_ref, v1_ref, v2_ref, br_ref, oe_ref, ou_ref):
        emb = (
            jnp.dot(x_ref[...], w_ref[...], preferred_element_type=jnp.float32,
                    precision=lax.Precision.HIGHEST)
            + b_ref[...]
        )
        u1 = jnp.dot(emb, v1_ref[...], preferred_element_type=jnp.float32,
                     precision=lax.Precision.HIGHEST) + br_ref[...]
        u2 = jnp.dot(emb, v2_ref[...], preferred_element_type=jnp.float32,
                     precision=lax.Precision.HIGHEST)
        oe_ref[...] = emb
        ou_ref[...] = jnp.concatenate([u1, u2], axis=1)

    return pl.pallas_call(
        body,
        grid=(N // BLK,),
        in_specs=[
            pl.BlockSpec((BLK, D), lambda i: (i, 0)),
            pl.BlockSpec((D, D), lambda i: (0, 0)),
            pl.BlockSpec((1, D), lambda i: (0, 0)),
            pl.BlockSpec((D, R), lambda i: (0, 0)),
            pl.BlockSpec((D, R), lambda i: (0, 0)),
            pl.BlockSpec((1, R), lambda i: (0, 0)),
        ],
        out_specs=[
            pl.BlockSpec((BLK, D), lambda i: (i, 0)),
            pl.BlockSpec((BLK, 2 * R), lambda i: (i, 0)),
        ],
        out_shape=[
            jax.ShapeDtypeStruct((N, D), jnp.float32),
            jax.ShapeDtypeStruct((N, 2 * R), jnp.float32),
        ],
    )(x, WaddT, badd_row, v1T, v2T, bias_row)


def _sc_score(emb_ext, w_relation, src_p, rel_p, dst_p, n_chunks):
    N = emb_ext.shape[0]
    R, D = w_relation.shape
    E_pad = src_p.shape[0]
    mesh = plsc.VectorSubcoreMesh(core_axis_name="c", subcore_axis_name="s")

    @functools.partial(
        pl.kernel,
        out_type=jax.ShapeDtypeStruct((E_pad,), jnp.float32),
        mesh=mesh,
        scratch_types=[
            pltpu.VMEM((2, _CH), jnp.int32),
            pltpu.VMEM((2, _CH), jnp.int32),
            pltpu.VMEM((2, _CH), jnp.int32),
            pltpu.VMEM((2, _CH, _RW), jnp.float32),
            pltpu.VMEM((2, _CH, _RW), jnp.float32),
            pltpu.VMEM((R, D), jnp.float32),
            pltpu.VMEM((2, _CH), jnp.float32),
            pltpu.SemaphoreType.DMA,
            pltpu.SemaphoreType.DMA,
            pltpu.SemaphoreType.DMA,
            pltpu.SemaphoreType.DMA,
        ],
    )
    def k(emb_hbm, wrel_hbm, src_hbm, rel_hbm, dst_hbm, out_hbm,
          src_v, dst_v, rel_v, s_rows, o_rows, wrel_v, score_v,
          sem_s0, sem_o0, sem_s1, sem_o1):
        wid = lax.axis_index("s") * _NC + lax.axis_index("c")
        pltpu.sync_copy(wrel_hbm, wrel_v)
        lanes = lax.iota(jnp.int32, _L)
        sems = ((sem_s0, sem_o0), (sem_s1, sem_o1))

        def issue(ph, ci):
            base = (wid * n_chunks + ci) * _CH
            pltpu.sync_copy(src_hbm.at[pl.ds(base, _CH)], src_v.at[ph])
            pltpu.sync_copy(dst_hbm.at[pl.ds(base, _CH)], dst_v.at[ph])
            pltpu.sync_copy(rel_hbm.at[pl.ds(base, _CH)], rel_v.at[ph])
            pltpu.async_copy(emb_hbm.at[src_v.at[ph]], s_rows.at[ph],
                             sems[ph][0])
            pltpu.async_copy(emb_hbm.at[dst_v.at[ph]], o_rows.at[ph],
                             sems[ph][1])

        def wait(ph):
            pltpu.make_async_copy(emb_hbm.at[src_v.at[ph]], s_rows.at[ph],
                                  sems[ph][0]).wait()
            pltpu.make_async_copy(emb_hbm.at[dst_v.at[ph]], o_rows.at[ph],
                                  sems[ph][1]).wait()

        def compute(ph, ci):
            base = (wid * n_chunks + ci) * _CH

            def group_body(g, carry2):
                score = jnp.zeros((_L,), jnp.float32)
                relg = rel_v[ph, pl.ds(g * _L, _L)]
                for t in range(_L):
                    row = g * _L + t
                    relt = relg[t]
                    u1 = s_rows[ph, row, pl.ds(D, _L)]
                    u2 = o_rows[ph, row, pl.ds(D + _L, _L)]
                    acc = jnp.where(lanes == relt, u1 + u2, 0.0)
                    for c in range(D // _L):
                        s = s_rows[ph, row, pl.ds(c * _L, _L)]
                        o = o_rows[ph, row, pl.ds(c * _L, _L)]
                        r = wrel_v[relt, pl.ds(c * _L, _L)]
                        acc = acc + s * r * o
                    tot = _lane_sum(acc)
                    score = jnp.where(lanes == t, tot, score)
                score_v[ph, pl.ds(g * _L, _L)] = score
                return carry2

            lax.fori_loop(0, _CH // _L, group_body, 0)
            pltpu.sync_copy(score_v.at[ph], out_hbm.at[pl.ds(base, _CH)])

        issue(0, 0)

        def pair_body(kk, carry):
            issue(1, 2 * kk + 1)
            wait(0)
            compute(0, 2 * kk)

            @pl.when(kk + 1 < n_chunks // 2)
            def _():
                issue(0, 2 * kk + 2)

            wait(1)
            compute(1, 2 * kk + 1)
            return carry

        lax.fori_loop(0, n_chunks // 2, pair_body, 0)

    return k(emb_ext, w_relation, src_p, rel_p, dst_p)


def kernel(x, W_add, b_add, w_relation, w_standard, bias, src, rel, dst):
    N, D = x.shape
    R = w_relation.shape[0]
    E = src.shape[0]

    emb, u = _tc_embed(
        x, W_add.T, b_add.reshape(1, D),
        w_standard[:, :D].T, w_standard[:, D:].T, bias.reshape(1, R),
    )
    pad_w = _RW - D - 2 * R
    emb_ext = jnp.concatenate(
        [emb, u, jnp.zeros((N, pad_w), jnp.float32)], axis=1)

    per_worker = -(-E // _NW)
    n_chunks = -(-per_worker // _CH)
    if n_chunks % 2:
        n_chunks += 1
    E_pad = _NW * n_chunks * _CH
    pad = E_pad - E
    src_p = jnp.pad(src, (0, pad))
    rel_p = jnp.pad(rel, (0, pad))
    dst_p = jnp.pad(dst, (0, pad))

    scores = _sc_score(emb_ext, w_relation, src_p, rel_p, dst_p, n_chunks)
    return scores[:E]
